# baseline (speedup 1.0000x reference)
import functools

import jax
import jax.numpy as jnp
from jax import lax
from jax.experimental import pallas as pl
from jax.experimental.pallas import tpu as pltpu
from jax.experimental.pallas import tpu_sc as plsc

_N = 10000
_E = 320000
_NC = 2
_NS = 16
_NW = _NC * _NS
_CH = 1024
_C = 10
_EP = _NW * _C * _CH
_NP = 10240
_RPT = _NP // _NS

_mesh = plsc.VectorSubcoreMesh(core_axis_name="c", subcore_axis_name="s")
_sc_params = pltpu.CompilerParams(use_tc_tiling_on_sc=False)


@functools.partial(
    pl.kernel,
    mesh=_mesh,
    out_type=jax.ShapeDtypeStruct((_NC, _NP), jnp.float32),
    compiler_params=_sc_params,
    scratch_types=[
        pltpu.VMEM((_C, _CH), jnp.int32),
        pltpu.VMEM((_CH,), jnp.float32),
        pltpu.VMEM_SHARED((_NP,), jnp.float32),
    ],
)
def _deg_partials(dsts, zeros1, out, didx, ones_v, acc):
    cid = lax.axis_index("c")
    sid = lax.axis_index("s")
    w = cid * _NS + sid
    r0 = sid * _RPT
    for i in range(_CH // 16):
        ones_v[pl.ds(i * 16, 16)] = jnp.ones((16,), jnp.float32)
    pltpu.sync_copy(zeros1.at[pl.ds(r0, _RPT)], acc.at[pl.ds(r0, _RPT)])
    pltpu.sync_copy(dsts.at[w], didx)
    plsc.subcore_barrier()

    def body(j, carry):
        pltpu.sync_copy(ones_v, acc.at[didx.at[j]], add=True)
        return carry

    lax.fori_loop(0, _C, body, 0)
    plsc.subcore_barrier()
    pltpu.sync_copy(acc.at[pl.ds(r0, _RPT)], out.at[cid, pl.ds(r0, _RPT)])


def _make_agg(D):
    @functools.partial(
        pl.kernel,
        mesh=_mesh,
        out_type=jax.ShapeDtypeStruct((_NC, _NP, D), jnp.float32),
        compiler_params=_sc_params,
        scratch_types=[
            pltpu.VMEM((_C, _CH), jnp.int32),
            pltpu.VMEM((_C, _CH), jnp.int32),
            pltpu.VMEM((_CH, D), jnp.float32),
            pltpu.SemaphoreType.DMA,
            pltpu.VMEM_SHARED((_NP, D), jnp.float32),
        ],
    )
    def agg(xw, srcs, dsts, zeros, out, sidx, didx, rows, sem, acc):
        cid = lax.axis_index("c")
        sid = lax.axis_index("s")
        w = cid * _NS + sid
        r0 = sid * _RPT
        pltpu.sync_copy(zeros.at[pl.ds(r0, _RPT)], acc.at[pl.ds(r0, _RPT)])
        pltpu.sync_copy(srcs.at[w], sidx)
        pltpu.sync_copy(dsts.at[w], didx)
        plsc.subcore_barrier()

        def body(j, carry):
            pltpu.async_copy(xw.at[sidx.at[j]], rows, sem).wait()
            pltpu.sync_copy(rows, acc.at[didx.at[j]], add=True)
            return carry

        lax.fori_loop(0, _C, body, 0)
        plsc.subcore_barrier()
        pltpu.sync_copy(acc.at[pl.ds(r0, _RPT)],
                        out.at[cid, pl.ds(r0, _RPT)])

    return agg


_agg64 = _make_agg(64)
_agg8 = _make_agg(8)

_BS = 1000


def _tc_xw1(x, W1, d0, d1):
    def body(x_r, w_r, d0_r, d1_r, o_r):
        dis = lax.rsqrt(d0_r[...] + d1_r[...] + 1.0)
        o_r[...] = jnp.dot(x_r[...], w_r[...],
                           preferred_element_type=jnp.float32) * dis

    return pl.pallas_call(
        body,
        grid=(_N // _BS,),
        in_specs=[
            pl.BlockSpec((_BS, 128), lambda i: (i, 0)),
            pl.BlockSpec((128, 64), lambda i: (0, 0)),
            pl.BlockSpec((_BS, 1), lambda i: (i, 0)),
            pl.BlockSpec((_BS, 1), lambda i: (i, 0)),
        ],
        out_specs=pl.BlockSpec((_BS, 64), lambda i: (i, 0)),
        out_shape=jax.ShapeDtypeStruct((_N, 64), jnp.float32),
    )(x, W1, d0, d1)


def _tc_mid(p0, p1, xw1p, d0, d1, W2, b1):
    def body(p0_r, p1_r, xw_r, d0_r, d1_r, w2_r, b1_r, o_r):
        dis = lax.rsqrt(d0_r[...] + d1_r[...] + 1.0)
        h = dis * (p0_r[...] + p1_r[...] + xw_r[...]) + b1_r[...]
        h = jnp.maximum(h, 0.0)
        o_r[...] = jnp.dot(h, w2_r[...],
                           preferred_element_type=jnp.float32) * dis

    return pl.pallas_call(
        body,
        grid=(_N // _BS,),
        in_specs=[
            pl.BlockSpec((_BS, 64), lambda i: (i, 0)),
            pl.BlockSpec((_BS, 64), lambda i: (i, 0)),
            pl.BlockSpec((_BS, 64), lambda i: (i, 0)),
            pl.BlockSpec((_BS, 1), lambda i: (i, 0)),
            pl.BlockSpec((_BS, 1), lambda i: (i, 0)),
            pl.BlockSpec((64, 8), lambda i: (0, 0)),
            pl.BlockSpec((1, 64), lambda i: (0, 0)),
        ],
        out_specs=pl.BlockSpec((_BS, 8), lambda i: (i, 0)),
        out_shape=jax.ShapeDtypeStruct((_N, 8), jnp.float32),
    )(p0, p1, xw1p, d0, d1, W2, b1)


def _tc_final(q0, q1, xw2p, d0, d1, b2):
    def body(q0_r, q1_r, xw_r, d0_r, d1_r, b2_r, o_r):
        dis = lax.rsqrt(d0_r[...] + d1_r[...] + 1.0)
        s = dis * (q0_r[...] + q1_r[...] + xw_r[...])
        o_r[...] = s[:, :2] + b2_r[...]

    return pl.pallas_call(
        body,
        grid=(_N // _BS,),
        in_specs=[
            pl.BlockSpec((_BS, 8), lambda i: (i, 0)),
            pl.BlockSpec((_BS, 8), lambda i: (i, 0)),
            pl.BlockSpec((_BS, 8), lambda i: (i, 0)),
            pl.BlockSpec((_BS, 1), lambda i: (i, 0)),
            pl.BlockSpec((_BS, 1), lambda i: (i, 0)),
            pl.BlockSpec((1, 2), lambda i: (0, 0)),
        ],
        out_specs=pl.BlockSpec((_BS, 2), lambda i: (i, 0)),
        out_shape=jax.ShapeDtypeStruct((_N, 2), jnp.float32),
    )(q0, q1, xw2p, d0, d1, b2)


def kernel(x, edge_index, W1, b1, W2, b2):
    src = edge_index[0]
    dst = edge_index[1]
    pad = _EP - _E
    srcs = jnp.concatenate(
        [src, jnp.zeros((pad,), jnp.int32)]).reshape(_NW, _C, _CH)
    dsts = jnp.concatenate(
        [dst, jnp.full((pad,), _N, jnp.int32)]).reshape(_NW, _C, _CH)
    z1 = jnp.zeros((_NP,), jnp.float32)
    z64 = jnp.zeros((_NP, 64), jnp.float32)
    z8 = jnp.zeros((_NP, 8), jnp.float32)
    W2p = jnp.zeros((64, 8), jnp.float32).at[:, :2].set(W2)

    degp = _deg_partials(dsts, z1)
    d0 = degp[0, :_N].reshape(_N, 1)
    d1 = degp[1, :_N].reshape(_N, 1)

    xw1p = _tc_xw1(x, W1, d0, d1)
    p = _agg64(xw1p, srcs, dsts, z64)
    xw2p = _tc_mid(p[0, :_N], p[1, :_N], xw1p, d0, d1,
                   W2p, b1.reshape(1, 64))
    q = _agg8(xw2p, srcs, dsts, z8)
    out = _tc_final(q[0, :_N], q[1, :_N], xw2p, d0, d1,
                    b2.reshape(1, 2))
    return out


# speedup vs baseline: 1.7629x; 1.7629x over previous
import functools

import jax
import jax.numpy as jnp
from jax import lax
from jax.experimental import pallas as pl
from jax.experimental.pallas import tpu as pltpu
from jax.experimental.pallas import tpu_sc as plsc

_N = 10000
_E = 320000
_NC = 2
_NS = 16
_NW = _NC * _NS
_CH = 1024
_C = 10
_EP = _NW * _C * _CH
_NP = 10240
_RPT = _NP // _NS

_mesh = plsc.VectorSubcoreMesh(core_axis_name="c", subcore_axis_name="s")
_sc_params = pltpu.CompilerParams(use_tc_tiling_on_sc=False)


@functools.partial(
    pl.kernel,
    mesh=_mesh,
    out_type=jax.ShapeDtypeStruct((_NC, _NP), jnp.float32),
    compiler_params=_sc_params,
    scratch_types=[
        pltpu.VMEM((_C, _CH), jnp.int32),
        pltpu.VMEM((_CH,), jnp.float32),
        pltpu.VMEM_SHARED((_NP,), jnp.float32),
    ],
)
def _deg_partials(dsts, zeros1, out, didx, ones_v, acc):
    cid = lax.axis_index("c")
    sid = lax.axis_index("s")
    w = cid * _NS + sid
    r0 = sid * _RPT
    for i in range(_CH // 16):
        ones_v[pl.ds(i * 16, 16)] = jnp.ones((16,), jnp.float32)
    pltpu.sync_copy(zeros1.at[pl.ds(r0, _RPT)], acc.at[pl.ds(r0, _RPT)])
    pltpu.sync_copy(dsts.at[w], didx)
    plsc.subcore_barrier()

    def body(j, carry):
        pltpu.sync_copy(ones_v, acc.at[didx.at[j]], add=True)
        return carry

    lax.fori_loop(0, _C, body, 0)
    plsc.subcore_barrier()
    pltpu.sync_copy(acc.at[pl.ds(r0, _RPT)], out.at[cid, pl.ds(r0, _RPT)])


def _make_agg(D):
    @functools.partial(
        pl.kernel,
        mesh=_mesh,
        out_type=jax.ShapeDtypeStruct((_NC, _NP, D), jnp.float32),
        compiler_params=_sc_params,
        scratch_types=[
            pltpu.VMEM((_C, _CH), jnp.int32),
            pltpu.VMEM((_C, _CH), jnp.int32),
            pltpu.VMEM((_CH, D), jnp.float32),
            pltpu.SemaphoreType.DMA,
            pltpu.VMEM_SHARED((_NP, D), jnp.float32),
            pltpu.VMEM_SHARED((_N, D), jnp.float32),
        ],
    )
    def agg(xw, srcs, dsts, zeros, out, sidx, didx, rows, sem, acc, table):
        cid = lax.axis_index("c")
        sid = lax.axis_index("s")
        w = cid * _NS + sid
        r0 = sid * _RPT
        t0 = sid * (_N // _NS)
        pltpu.sync_copy(xw.at[pl.ds(t0, _N // _NS)],
                        table.at[pl.ds(t0, _N // _NS)])
        pltpu.sync_copy(zeros.at[pl.ds(r0, _RPT)], acc.at[pl.ds(r0, _RPT)])
        pltpu.sync_copy(srcs.at[w], sidx)
        pltpu.sync_copy(dsts.at[w], didx)
        plsc.subcore_barrier()

        def body(j, carry):
            pltpu.async_copy(table.at[sidx.at[j]], rows, sem).wait()
            pltpu.sync_copy(rows, acc.at[didx.at[j]], add=True)
            return carry

        lax.fori_loop(0, _C, body, 0)
        plsc.subcore_barrier()
        pltpu.sync_copy(acc.at[pl.ds(r0, _RPT)],
                        out.at[cid, pl.ds(r0, _RPT)])

    return agg


_agg32 = _make_agg(32)
_agg8 = _make_agg(8)

_BS = 1000


def _tc_xw1(x, W1, d0, d1):
    def body(x_r, w_r, d0_r, d1_r, oa_r, ob_r):
        dis = lax.rsqrt(d0_r[...] + d1_r[...] + 1.0)
        xw = jnp.dot(x_r[...], w_r[...],
                     preferred_element_type=jnp.float32) * dis
        oa_r[...] = xw[:, :32]
        ob_r[...] = xw[:, 32:]

    return pl.pallas_call(
        body,
        grid=(_N // _BS,),
        in_specs=[
            pl.BlockSpec((_BS, 128), lambda i: (i, 0)),
            pl.BlockSpec((128, 64), lambda i: (0, 0)),
            pl.BlockSpec((_BS, 1), lambda i: (i, 0)),
            pl.BlockSpec((_BS, 1), lambda i: (i, 0)),
        ],
        out_specs=[pl.BlockSpec((_BS, 32), lambda i: (i, 0)),
                   pl.BlockSpec((_BS, 32), lambda i: (i, 0))],
        out_shape=[jax.ShapeDtypeStruct((_N, 32), jnp.float32),
                   jax.ShapeDtypeStruct((_N, 32), jnp.float32)],
    )(x, W1, d0, d1)


def _tc_mid(pa0, pa1, pb0, pb1, xwa, xwb, d0, d1, W2, b1):
    def body(pa0_r, pa1_r, pb0_r, pb1_r, xwa_r, xwb_r,
             d0_r, d1_r, w2_r, b1_r, o_r):
        dis = lax.rsqrt(d0_r[...] + d1_r[...] + 1.0)
        tot = jnp.concatenate(
            [pa0_r[...] + pa1_r[...] + xwa_r[...],
             pb0_r[...] + pb1_r[...] + xwb_r[...]], axis=1)
        h = jnp.maximum(dis * tot + b1_r[...], 0.0)
        o_r[...] = jnp.dot(h, w2_r[...],
                           preferred_element_type=jnp.float32) * dis

    return pl.pallas_call(
        body,
        grid=(_N // _BS,),
        in_specs=[
            pl.BlockSpec((_BS, 32), lambda i: (i, 0)),
            pl.BlockSpec((_BS, 32), lambda i: (i, 0)),
            pl.BlockSpec((_BS, 32), lambda i: (i, 0)),
            pl.BlockSpec((_BS, 32), lambda i: (i, 0)),
            pl.BlockSpec((_BS, 32), lambda i: (i, 0)),
            pl.BlockSpec((_BS, 32), lambda i: (i, 0)),
            pl.BlockSpec((_BS, 1), lambda i: (i, 0)),
            pl.BlockSpec((_BS, 1), lambda i: (i, 0)),
            pl.BlockSpec((64, 8), lambda i: (0, 0)),
            pl.BlockSpec((1, 64), lambda i: (0, 0)),
        ],
        out_specs=pl.BlockSpec((_BS, 8), lambda i: (i, 0)),
        out_shape=jax.ShapeDtypeStruct((_N, 8), jnp.float32),
    )(pa0, pa1, pb0, pb1, xwa, xwb, d0, d1, W2, b1)


def _tc_final(q0, q1, xw2p, d0, d1, b2):
    def body(q0_r, q1_r, xw_r, d0_r, d1_r, b2_r, o_r):
        dis = lax.rsqrt(d0_r[...] + d1_r[...] + 1.0)
        s = dis * (q0_r[...] + q1_r[...] + xw_r[...])
        o_r[...] = s[:, :2] + b2_r[...]

    return pl.pallas_call(
        body,
        grid=(_N // _BS,),
        in_specs=[
            pl.BlockSpec((_BS, 8), lambda i: (i, 0)),
            pl.BlockSpec((_BS, 8), lambda i: (i, 0)),
            pl.BlockSpec((_BS, 8), lambda i: (i, 0)),
            pl.BlockSpec((_BS, 1), lambda i: (i, 0)),
            pl.BlockSpec((_BS, 1), lambda i: (i, 0)),
            pl.BlockSpec((1, 2), lambda i: (0, 0)),
        ],
        out_specs=pl.BlockSpec((_BS, 2), lambda i: (i, 0)),
        out_shape=jax.ShapeDtypeStruct((_N, 2), jnp.float32),
    )(q0, q1, xw2p, d0, d1, b2)


def kernel(x, edge_index, W1, b1, W2, b2):
    src = edge_index[0]
    dst = edge_index[1]
    pad = _EP - _E
    srcs = jnp.concatenate(
        [src, jnp.zeros((pad,), jnp.int32)]).reshape(_NW, _C, _CH)
    dsts = jnp.concatenate(
        [dst, jnp.full((pad,), _N, jnp.int32)]).reshape(_NW, _C, _CH)
    z1 = jnp.zeros((_NP,), jnp.float32)
    z32 = jnp.zeros((_NP, 32), jnp.float32)
    z8 = jnp.zeros((_NP, 8), jnp.float32)
    W2p = jnp.zeros((64, 8), jnp.float32).at[:, :2].set(W2)

    degp = _deg_partials(dsts, z1)
    d0 = degp[0, :_N].reshape(_N, 1)
    d1 = degp[1, :_N].reshape(_N, 1)

    xwa, xwb = _tc_xw1(x, W1, d0, d1)
    pa = _agg32(xwa, srcs, dsts, z32)
    pb = _agg32(xwb, srcs, dsts, z32)
    xw2p = _tc_mid(pa[0, :_N], pa[1, :_N], pb[0, :_N], pb[1, :_N],
                   xwa, xwb, d0, d1,
                   W2p, b1.reshape(1, 64))
    q = _agg8(xw2p, srcs, dsts, z8)
    out = _tc_final(q[0, :_N], q[1, :_N], xw2p, d0, d1,
                    b2.reshape(1, 2))
    return out
